# Initial kernel scaffold; baseline (speedup 1.0000x reference)
import jax
import jax.numpy as jnp
from jax.experimental import pallas as pl


def kernel(p, x, b, sa1_params, sa2_params, sa3_params, fp3_params, fp2_params, fp1_params, head_W, head_b):
    raise NotImplementedError("write your pallas kernel here")


# speedup vs baseline: 3.7285x; 3.7285x over previous
import functools

import jax
import jax.numpy as jnp
from jax import lax
from jax.experimental import pallas as pl
from jax.experimental.pallas import tpu as pltpu
from jax.experimental.pallas import tpu_sc as plsc

_EPS = 1e-5
_BIG = 1e30
_QPAD = 1e4
_SPAD = -1e4


def _ceil_to(v, m):
  return (v + m - 1) // m * m


def _bn_apply(h, mean, var, g, be):
  return g * (h - mean) / jnp.sqrt(var + _EPS) + be


def _fps_body(px_ref, py_ref, pz_ref, qx_ref, qy_ref, qz_ref, dist_ref, *, m):
  R = px_ref.shape[0]
  rows = lax.broadcasted_iota(jnp.int32, (R, 128), 0)
  cols = lax.broadcasted_iota(jnp.int32, (R, 128), 1)
  flat = rows * 128 + cols
  px = px_ref[...]
  py = py_ref[...]
  pz = pz_ref[...]
  dist_ref[...] = jnp.full((R, 128), _BIG, jnp.float32)

  def coord(arr, nxt):
    return jnp.sum(jnp.where(flat == nxt, arr, 0.0))

  def bc(v):
    return jnp.zeros((1, 128), jnp.float32) + v

  qx0 = coord(px, 0)
  qy0 = coord(py, 0)
  qz0 = coord(pz, 0)
  qx_ref[pl.ds(0, 1), :] = bc(qx0)
  qy_ref[pl.ds(0, 1), :] = bc(qy0)
  qz_ref[pl.ds(0, 1), :] = bc(qz0)

  def step(i, carry):
    qx, qy, qz = carry
    d = (px - qx) ** 2 + (py - qy) ** 2 + (pz - qz) ** 2
    dm = jnp.minimum(dist_ref[...], d)
    dist_ref[...] = dm
    mx = jnp.max(dm)
    nxt = jnp.min(jnp.where(dm == mx, flat, jnp.int32(2 ** 30)))
    nqx = coord(px, nxt)
    nqy = coord(py, nxt)
    nqz = coord(pz, nxt)
    qx_ref[pl.ds(i, 1), :] = bc(nqx)
    qy_ref[pl.ds(i, 1), :] = bc(nqy)
    qz_ref[pl.ds(i, 1), :] = bc(nqz)
    return (nqx, nqy, nqz)

  lax.fori_loop(1, m, step, (qx0, qy0, qz0))


def _fps(p, m):
  n = p.shape[0]
  npad = _ceil_to(n, 128)
  R = npad // 128
  ppad = jnp.concatenate([p, jnp.broadcast_to(p[0:1], (npad - n, 3))], axis=0)
  cxyz = [ppad[:, i].reshape(R, 128) for i in range(3)]
  mpad = _ceil_to(m, 8)
  out_sd = jax.ShapeDtypeStruct((mpad, 128), jnp.float32)
  qx, qy, qz = pl.pallas_call(
      functools.partial(_fps_body, m=m),
      out_shape=(out_sd, out_sd, out_sd),
      scratch_shapes=[pltpu.VMEM((R, 128), jnp.float32)],
  )(*cxyz)
  return jnp.stack([qx[:m, 0], qy[:m, 0], qz[:m, 0]], axis=1)


def _radius_body(qx_ref, qy_ref, qz_ref, sx_ref, sy_ref, sz_ref,
                 nbr_ref, mf_ref, d2_ref, *, r2, nsrc, k, cs):
  S = sx_ref.shape[0]
  nch = S // cs
  qx = qx_ref[...]
  qy = qy_ref[...]
  qz = qz_ref[...]

  rowi = lax.broadcasted_iota(jnp.int32, (cs, 128), 0)

  def pass_j(j, prev):
    def chunk(c, carry):
      gmn, gidx = carry
      off = c * cs
      sx = sx_ref[pl.ds(off, cs), :]
      sy = sy_ref[pl.ds(off, cs), :]
      sz = sz_ref[pl.ds(off, cs), :]
      rid = rowi + off

      @pl.when(j == 0)
      def _():
        d2 = (qx - sx) ** 2 + (qy - sy) ** 2 + (qz - sz) ** 2
        d2_ref[pl.ds(off, cs), :] = d2

      v = d2_ref[pl.ds(off, cs), :]
      v = jnp.where(rid == prev, _BIG, v)
      d2_ref[pl.ds(off, cs), :] = v
      cmn = jnp.min(v, axis=0, keepdims=True)
      cidx = jnp.min(jnp.where(v == cmn, rid, jnp.int32(2 ** 30)),
                     axis=0, keepdims=True)
      upd = cmn < gmn
      return (jnp.where(upd, cmn, gmn), jnp.where(upd, cidx, gidx))

    gmn0 = jnp.full((1, 128), _BIG, jnp.float32)
    gidx0 = jnp.full((1, 128), 2 ** 30, jnp.int32)
    gmn, gidx = lax.fori_loop(0, nch, chunk, (gmn0, gidx0))
    nbr_ref[pl.ds(j, 1), :] = jnp.minimum(gidx, nsrc - 1)
    mf_ref[pl.ds(j, 1), :] = (gmn <= r2).astype(jnp.float32)
    return gidx

  lax.fori_loop(0, k, pass_j, jnp.full((1, 128), -1, jnp.int32))


def _radius(pos_src, pos_q, r, k, nsrc_real):
  S = pos_src.shape[0]
  Q = pos_q.shape[0]
  cs = min(S, 512)
  sx = pos_src[:, 0].reshape(S, 1)
  sy = pos_src[:, 1].reshape(S, 1)
  sz = pos_src[:, 2].reshape(S, 1)
  qx = pos_q[:, 0].reshape(1, Q)
  qy = pos_q[:, 1].reshape(1, Q)
  qz = pos_q[:, 2].reshape(1, Q)
  nqt = Q // 128
  grid = (nqt,)
  qspec = pl.BlockSpec((1, 128), lambda i: (0, i))
  sspec = pl.BlockSpec((S, 1), lambda i: (0, 0))
  ospec = pl.BlockSpec((k, 128), lambda i: (0, i))
  nbr_t, mf_t = pl.pallas_call(
      functools.partial(_radius_body, r2=float(r) * float(r), nsrc=nsrc_real,
                        k=k, cs=cs),
      grid=grid,
      in_specs=[qspec, qspec, qspec, sspec, sspec, sspec],
      out_specs=(ospec, ospec),
      out_shape=(jax.ShapeDtypeStruct((k, Q), jnp.int32),
                 jax.ShapeDtypeStruct((k, Q), jnp.float32)),
      scratch_shapes=[pltpu.VMEM((S, 128), jnp.float32)],
  )(qx, qy, qz, sx, sy, sz)
  return nbr_t.T, mf_t.T


def _sc_gather(table, idx, chunk=128):
  info = plsc.get_sparse_core_info()
  nw = info.num_cores * info.num_subcores
  B = idx.shape[0]
  D = table.shape[1]
  bw = B // nw
  nch = bw // chunk
  mesh = plsc.VectorSubcoreMesh(core_axis_name="c", subcore_axis_name="s")

  @functools.partial(
      pl.kernel,
      out_type=jax.ShapeDtypeStruct((B, D), jnp.float32),
      mesh=mesh,
      scratch_types=[
          pltpu.VMEM((chunk,), jnp.int32),
          pltpu.VMEM((chunk, D), jnp.float32),
          pltpu.SemaphoreType.DMA,
      ],
  )
  def k(table_hbm, idx_hbm, out_hbm, idx_v, rows_v, sem):
    wid = lax.axis_index("s") * info.num_cores + lax.axis_index("c")
    base = wid * bw

    def body(j, _):
      off = base + j * chunk
      pltpu.sync_copy(idx_hbm.at[pl.ds(off, chunk)], idx_v)
      pltpu.async_copy(table_hbm.at[idx_v], rows_v, sem).wait()
      pltpu.sync_copy(rows_v, out_hbm.at[pl.ds(off, chunk)])
      return 0

    lax.fori_loop(0, nch, body, 0, unroll=False)

  return k(table, idx)


def _sum_stats(h, mf, tc):
  s0 = jnp.sum(jnp.where(mf > 0, h, 0.0), axis=0, keepdims=True)
  cnt = jnp.zeros((1, tc), jnp.float32) + jnp.sum(mf)
  return jnp.concatenate(
      [s0, cnt, jnp.zeros((6, tc), jnp.float32)], axis=0)


def _layer_body(x_ref, mean_ref, var_ref, g_ref, be_ref, w_ref, b_ref,
                mf_ref, h_ref, st_ref, *, norm):
  i = pl.program_id(1)
  tc = w_ref.shape[1]
  xx = x_ref[...]
  if norm:
    xx = _bn_apply(xx, mean_ref[...], var_ref[...], g_ref[...], be_ref[...])
  h = jnp.dot(xx, w_ref[...], preferred_element_type=jnp.float32) + b_ref[...]
  h = jnp.maximum(h, 0.0)
  h_ref[...] = h
  part = _sum_stats(h, mf_ref[...], tc)

  @pl.when(i == 0)
  def _():
    st_ref[...] = part

  @pl.when(i > 0)
  def _():
    st_ref[...] += part


def _layer_mm(x, prior, w, b, mf, tr=512, tc=512):
  rows, cin = x.shape
  cout = w.shape[1]
  tr = min(tr, rows)
  tc = min(tc, cout)
  grid = (cout // tc, rows // tr)
  norm = prior is not None
  if norm:
    mean, var, g, be = prior
  else:
    mean = jnp.zeros((1, cin), jnp.float32)
    var = jnp.ones((1, cin), jnp.float32)
    g = jnp.ones((1, cin), jnp.float32)
    be = jnp.zeros((1, cin), jnp.float32)
  vspec = pl.BlockSpec((1, cin), lambda j, i: (0, 0))
  h, st = pl.pallas_call(
      functools.partial(_layer_body, norm=norm),
      grid=grid,
      in_specs=[
          pl.BlockSpec((tr, cin), lambda j, i: (i, 0)),
          vspec, vspec, vspec, vspec,
          pl.BlockSpec((cin, tc), lambda j, i: (0, j)),
          pl.BlockSpec((1, tc), lambda j, i: (0, j)),
          pl.BlockSpec((tr, 1), lambda j, i: (i, 0)),
      ],
      out_specs=(pl.BlockSpec((tr, tc), lambda j, i: (i, j)),
                 pl.BlockSpec((8, tc), lambda j, i: (0, j))),
      out_shape=(jax.ShapeDtypeStruct((rows, cout), jnp.float32),
                 jax.ShapeDtypeStruct((8, cout), jnp.float32)),
  )(x, mean, var, g, be, w, b, mf)
  return h, st


def _var_body(h_ref, mean_ref, mf_ref, st_ref):
  i = pl.program_id(1)
  tc = h_ref.shape[1]
  d = h_ref[...] - mean_ref[...]
  s = jnp.sum(jnp.where(mf_ref[...] > 0, d * d, 0.0), axis=0, keepdims=True)
  part = jnp.concatenate([s, jnp.zeros((7, tc), jnp.float32)], axis=0)

  @pl.when(i == 0)
  def _():
    st_ref[...] = part

  @pl.when(i > 0)
  def _():
    st_ref[...] += part


def _var_pass(h, mean, mf, tr=512, tc=512):
  rows, cout = h.shape
  tr = min(tr, rows)
  tc = min(tc, cout)
  grid = (cout // tc, rows // tr)
  st = pl.pallas_call(
      _var_body,
      grid=grid,
      in_specs=[
          pl.BlockSpec((tr, tc), lambda j, i: (i, j)),
          pl.BlockSpec((1, tc), lambda j, i: (0, j)),
          pl.BlockSpec((tr, 1), lambda j, i: (i, 0)),
      ],
      out_specs=pl.BlockSpec((8, tc), lambda j, i: (0, j)),
      out_shape=jax.ShapeDtypeStruct((8, cout), jnp.float32),
  )(h, mean, mf)
  return st


def _layer_full(x, prior, lparams, mf):
  w, b, g, be = lparams
  wp = _pad_rows(w, x.shape[1], 0.0)
  h, st = _layer_mm(x, prior, wp, b.reshape(1, -1), mf)
  cnt = jnp.maximum(st[1:2, 0:1], 1.0)
  mean = st[0:1] / cnt
  vs = _var_pass(h, mean, mf)
  var = vs[0:1] / cnt
  return h, (mean, var, g.reshape(1, -1), be.reshape(1, -1))


def _mlp_chain(x0, params, mf, prior=None):
  h = x0
  for lp in params:
    h, prior = _layer_full(h, prior, lp, mf)
  return h, prior


def _edge_l1_body(g_ref, pq_ref, w_ref, b_ref, mf_ref, h_ref, st_ref, *,
                  xdim):
  i = pl.program_id(0)
  cin = g_ref.shape[1]
  tc = w_ref.shape[1]
  nq = pq_ref.shape[0]
  rows = g_ref.shape[0]
  pqe = jnp.broadcast_to(pq_ref[...].reshape(nq, 1, 8),
                         (nq, 64, 8)).reshape(rows, 8)
  parts = [jnp.zeros((rows, xdim), jnp.float32), pqe[:, 0:3]]
  if cin > xdim + 3:
    parts.append(jnp.zeros((rows, cin - xdim - 3), jnp.float32))
  sub = jnp.concatenate(parts, axis=1)
  x = g_ref[...] - sub
  h = jnp.dot(x, w_ref[...], preferred_element_type=jnp.float32) + b_ref[...]
  h = jnp.maximum(h, 0.0)
  h_ref[...] = h
  part = _sum_stats(h, mf_ref[...], tc)

  @pl.when(i == 0)
  def _():
    st_ref[...] = part

  @pl.when(i > 0)
  def _():
    st_ref[...] += part


def _edge_l1(g, pq8, w, b, mf, xdim):
  rows, cin = g.shape
  cout = w.shape[1]
  nq = 8
  tr = nq * 64
  grid = (rows // tr,)
  h, st = pl.pallas_call(
      functools.partial(_edge_l1_body, xdim=xdim),
      grid=grid,
      in_specs=[
          pl.BlockSpec((tr, cin), lambda i: (i, 0)),
          pl.BlockSpec((nq, 8), lambda i: (i, 0)),
          pl.BlockSpec((cin, cout), lambda i: (0, 0)),
          pl.BlockSpec((1, cout), lambda i: (0, 0)),
          pl.BlockSpec((tr, 1), lambda i: (i, 0)),
      ],
      out_specs=(pl.BlockSpec((tr, cout), lambda i: (i, 0)),
                 pl.BlockSpec((8, cout), lambda i: (0, 0))),
      out_shape=(jax.ShapeDtypeStruct((rows, cout), jnp.float32),
                 jax.ShapeDtypeStruct((8, cout), jnp.float32)),
  )(g, pq8, w, b, mf)
  cnt = jnp.maximum(st[1:2, 0:1], 1.0)
  mean = st[0:1] / cnt
  vs = _var_pass(h, mean, mf)
  var = vs[0:1] / cnt
  return h, mean, var, cnt


def _segmax_body(h_ref, mean_ref, var_ref, g_ref, be_ref, mf_ref, o_ref, *,
                 seg, nq):
  tc = h_ref.shape[1]
  hn = _bn_apply(h_ref[...], mean_ref[...], var_ref[...], g_ref[...],
                 be_ref[...])
  mf = mf_ref[...]
  hm = jnp.where(mf > 0, hn, -_BIG)
  mx = jnp.max(hm.reshape(nq, seg, tc), axis=1)
  valid = jnp.max(mf.reshape(nq, seg, 1), axis=1)
  o_ref[...] = jnp.where(valid > 0, mx, 0.0)


def _segmax(h, prior, mf, seg):
  mean, var, g, be = prior
  rows, cout = h.shape
  nseg = rows // seg
  nq = min(nseg, max(1, 512 // seg))
  tr = nq * seg
  grid = (rows // tr,)
  vspec = pl.BlockSpec((1, cout), lambda i: (0, 0))
  return pl.pallas_call(
      functools.partial(_segmax_body, seg=seg, nq=nq),
      grid=grid,
      in_specs=[
          pl.BlockSpec((tr, cout), lambda i: (i, 0)),
          vspec, vspec, vspec, vspec,
          pl.BlockSpec((tr, 1), lambda i: (i, 0)),
      ],
      out_specs=pl.BlockSpec((nq, cout), lambda i: (i, 0)),
      out_shape=jax.ShapeDtypeStruct((nseg, cout), jnp.float32),
  )(h, mean, var, g, be, mf)


def _head_body(x_ref, mean_ref, var_ref, g_ref, be_ref, w_ref, b_ref, o_ref):
  xn = _bn_apply(x_ref[...], mean_ref[...], var_ref[...], g_ref[...],
                 be_ref[...])
  o_ref[...] = (jnp.dot(xn, w_ref[...], preferred_element_type=jnp.float32)
                + b_ref[...])


def _head(x, prior, w, b, tr=512):
  mean, var, g, be = prior
  rows, cin = x.shape
  cout = w.shape[1]
  tr = min(tr, rows)
  grid = (rows // tr,)
  vspec = pl.BlockSpec((1, cin), lambda i: (0, 0))
  return pl.pallas_call(
      _head_body,
      grid=grid,
      in_specs=[
          pl.BlockSpec((tr, cin), lambda i: (i, 0)),
          vspec, vspec, vspec, vspec,
          pl.BlockSpec((cin, cout), lambda i: (0, 0)),
          pl.BlockSpec((1, cout), lambda i: (0, 0)),
      ],
      out_specs=pl.BlockSpec((tr, cout), lambda i: (i, 0)),
      out_shape=jax.ShapeDtypeStruct((rows, cout), jnp.float32),
  )(x, mean, var, g, be, w, b)


def _knn3_body(dx_ref, dy_ref, dz_ref, sx_ref, sy_ref, sz_ref,
               fx_ref, mean_ref, var_ref, g_ref, be_ref, rm_ref, o_ref):
  S = sx_ref.shape[1]
  T = dx_ref.shape[0]
  dx = dx_ref[...]
  dy = dy_ref[...]
  dz = dz_ref[...]
  sx = sx_ref[...]
  sy = sy_ref[...]
  sz = sz_ref[...]
  d2 = (dx - sx) ** 2 + (dy - sy) ** 2 + (dz - sz) ** 2
  cols = lax.broadcasted_iota(jnp.int32, (T, S), 1)
  fxn = jnp.where(rm_ref[...] > 0,
                  _bn_apply(fx_ref[...], mean_ref[...], var_ref[...],
                            g_ref[...], be_ref[...]),
                  0.0)
  rows_k = []
  ws = []
  for _ in range(3):
    mn = jnp.min(d2, axis=1, keepdims=True)
    sel = jnp.min(jnp.where(d2 == mn, cols, jnp.int32(2 ** 30)),
                  axis=1, keepdims=True)
    oh = (cols == sel)
    d2 = jnp.where(oh, _BIG, d2)
    sxk = jnp.sum(jnp.where(oh, sx, 0.0), axis=1, keepdims=True)
    syk = jnp.sum(jnp.where(oh, sy, 0.0), axis=1, keepdims=True)
    szk = jnp.sum(jnp.where(oh, sz, 0.0), axis=1, keepdims=True)
    d2k = (dx - sxk) ** 2 + (dy - syk) ** 2 + (dz - szk) ** 2
    ws.append(1.0 / jnp.maximum(d2k, 1e-16))
    rows_k.append(jnp.dot(oh.astype(jnp.float32), fxn,
                          preferred_element_type=jnp.float32))
  wsum = ws[0] + ws[1] + ws[2]
  o_ref[...] = ((ws[0] / wsum) * rows_k[0] + (ws[1] / wsum) * rows_k[1]
                + (ws[2] / wsum) * rows_k[2])


def _knn3(pos_dst, pos_src, fx, prior, rm, tr=256):
  mean, var, g, be = prior
  T = pos_dst.shape[0]
  S = pos_src.shape[0]
  C = fx.shape[1]
  tr = min(tr, T)
  dx = pos_dst[:, 0].reshape(T, 1)
  dy = pos_dst[:, 1].reshape(T, 1)
  dz = pos_dst[:, 2].reshape(T, 1)
  sx = pos_src[:, 0].reshape(1, S)
  sy = pos_src[:, 1].reshape(1, S)
  sz = pos_src[:, 2].reshape(1, S)
  grid = (T // tr,)
  dspec = pl.BlockSpec((tr, 1), lambda i: (i, 0))
  sspec = pl.BlockSpec((1, S), lambda i: (0, 0))
  cspec = pl.BlockSpec((1, C), lambda i: (0, 0))
  return pl.pallas_call(
      _knn3_body,
      grid=grid,
      in_specs=[
          dspec, dspec, dspec, sspec, sspec, sspec,
          pl.BlockSpec((S, C), lambda i: (0, 0)),
          cspec, cspec, cspec, cspec,
          pl.BlockSpec((S, 1), lambda i: (0, 0)),
      ],
      out_specs=pl.BlockSpec((tr, C), lambda i: (i, 0)),
      out_shape=jax.ShapeDtypeStruct((T, C), jnp.float32),
  )(dx, dy, dz, sx, sy, sz, fx, mean, var, g, be, rm)


def _pad_rows(arr, rows, value):
  n = arr.shape[0]
  if rows == n:
    return arr
  pad = jnp.full((rows - n,) + arr.shape[1:], value, arr.dtype)
  return jnp.concatenate([arr, pad], axis=0)


def _pad_cols(arr, cols, value=0.0):
  c = arr.shape[1]
  if cols == c:
    return arr
  pad = jnp.full((arr.shape[0], cols - c), value, arr.dtype)
  return jnp.concatenate([arr, pad], axis=1)


def _row_mask(rows, nreal):
  return (jnp.arange(rows, dtype=jnp.int32) < nreal
          ).astype(jnp.float32).reshape(rows, 1)


def _sa(x_feat, pos_pad, pos_real_n, m, r, params, qpadto):
  pos_real = pos_pad[:pos_real_n]
  pos_q = _fps(pos_real, m)
  pos_q_pad = _pad_rows(pos_q, qpadto, _QPAD)
  nbr, mf = _radius(pos_pad, pos_q_pad, r, 64, pos_real_n)
  src = nbr.reshape(-1)
  mf_e = mf.reshape(-1, 1)

  (w1, b1, g1, be1) = params[0]
  cf = x_feat.shape[1]
  cin_pad = _ceil_to(cf + 3, 128)
  table = _pad_cols(jnp.concatenate([x_feat, pos_pad], axis=1), cin_pad)
  qp8 = _pad_cols(pos_q_pad, 8)

  g_rows = _sc_gather(table, src)[:, :cf + 3]
  h, mean, var, _ = _edge_l1(g_rows, qp8, w1, b1.reshape(1, -1), mf_e,
                             xdim=cf)
  prior = (mean, var, g1.reshape(1, -1), be1.reshape(1, -1))
  h, prior = _mlp_chain(h, params[1:], mf_e, prior=prior)
  sa_x = _segmax(h, prior, mf_e, 64)
  return sa_x, pos_q, pos_q_pad


def kernel(p, x, b, sa1_params, sa2_params, sa3_params, fp3_params,
           fp2_params, fp1_params, head_W, head_b):
  del b
  n = p.shape[0]
  m1 = max(1, int(n * 0.1))
  m2 = max(1, int(m1 * 0.05))
  npad = _ceil_to(n, 512)
  m1pad = _ceil_to(m1, 128)
  m2pad = 128

  p_pad = jnp.where(jnp.arange(npad)[:, None] < n,
                    _pad_rows(p, npad, 0.0),
                    jnp.broadcast_to(p[0:1], (npad, 3)))
  p_spad = _pad_rows(p, npad, _SPAD)
  x_pad = _pad_rows(x, npad, 0.0)

  sa1_x, sa1_pq, _ = _sa(x_pad, p_spad, n, m1, 0.2, sa1_params, m1pad)
  sa1_p_spad = _pad_rows(sa1_pq, m1pad, _SPAD)

  sa2_x, sa2_pq, _ = _sa(sa1_x, sa1_p_spad, m1, m2, 0.4, sa2_params, m2pad)

  rm2 = _row_mask(m2pad, m2)
  x3 = jnp.concatenate([sa2_x, _pad_rows(sa2_pq, m2pad, _QPAD)], axis=1)
  h3, pr3 = _mlp_chain(x3, sa3_params, rm2)
  sa3_x = _segmax(h3, pr3, rm2, m2pad)

  xf3 = jnp.concatenate([jnp.broadcast_to(sa3_x, (m2pad, sa3_x.shape[1])),
                         sa2_x], axis=1)
  hf3, prf3 = _mlp_chain(xf3, fp3_params, rm2)

  rm1 = _row_mask(m1pad, m1)
  i2 = _knn3(sa1_p_spad, _pad_rows(sa2_pq, m2pad, _QPAD), hf3, prf3, rm2)
  xf2 = jnp.concatenate([i2, sa1_x], axis=1)
  hf2, prf2 = _mlp_chain(xf2, fp2_params, rm1)

  rmn = _row_mask(npad, n)
  i1 = _knn3(p_pad, sa1_p_spad, hf2, prf2, rm1)
  xf1 = jnp.concatenate([i1, x_pad], axis=1)
  hf1, prf1 = _mlp_chain(xf1, fp1_params, rmn)

  out = _head(hf1, prf1, head_W, head_b.reshape(1, -1))
  return out[:n]
